# Initial kernel scaffold; baseline (speedup 1.0000x reference)
import jax
import jax.numpy as jnp
from jax.experimental import pallas as pl


def kernel(point, edge_info, W1, a_src1, a_dst1, b1, W2, a_src2, a_dst2, b2):
    raise NotImplementedError("write your pallas kernel here")


# speedup vs baseline: 82.2103x; 82.2103x over previous
import functools

import numpy as np
import jax
import jax.numpy as jnp
from jax import lax
from jax.experimental import pallas as pl
from jax.experimental.pallas import tpu as pltpu
from jax.experimental.pallas import tpu_sc as plsc

_N = 10000
_E = 320000
_D_IN = 128
_HID = 16
_OUT = 8
_HEADS = 8

_NC = 2
_NS = 16
_NW = _NC * _NS
_CH = 128
_NCHUNKS = _E // _CH
_RPT = 632
_NP = _NS * _RPT

_F1 = _D_IN + 16
_F2 = _HEADS * _OUT + 16


def _make_edge_kernel(F, c_per_head):
    G = F // 16
    shift = 4 if c_per_head == 16 else 3
    mesh = plsc.VectorSubcoreMesh(core_axis_name="c", subcore_axis_name="s")

    @functools.partial(
        pl.kernel,
        mesh=mesh,
        compiler_params=pltpu.CompilerParams(use_tc_tiling_on_sc=False),
        out_type=jax.ShapeDtypeStruct((_NC, _NP, F), jnp.float32),
        scratch_types=[
            pltpu.VMEM((_CH,), jnp.int32),
            pltpu.VMEM((_CH,), jnp.int32),
            pltpu.VMEM((_CH, F), jnp.float32),
            pltpu.VMEM((_CH, 16), jnp.float32),
            pltpu.VMEM_SHARED((_NP, F), jnp.float32),
            pltpu.SemaphoreType.DMA,
            pltpu.SemaphoreType.DMA,
        ],
    )
    def edge_kernel(hext, adt, edges, zrows, out,
                    sidx, didx, hbuf, adbuf, acc, sem1, sem2):
        c = lax.axis_index("c")
        s = lax.axis_index("s")
        wid = c * _NS + s
        pltpu.sync_copy(zrows, acc.at[pl.ds(s * _RPT, _RPT)])
        plsc.subcore_barrier()

        lane = lax.iota(jnp.int32, 16)
        nk = (_NCHUNKS + _NW - 1 - wid) // _NW

        def chunk_body(k, carry):
            base = (k * _NW + wid) * _CH
            pltpu.sync_copy(edges.at[0, pl.ds(base, _CH)], sidx)
            pltpu.sync_copy(edges.at[1, pl.ds(base, _CH)], didx)
            cp1 = pltpu.async_copy(hext.at[sidx], hbuf, sem1)
            cp2 = pltpu.async_copy(adt.at[didx], adbuf, sem2)
            cp1.wait()
            cp2.wait()

            def edge_body(i, carry2):
                e = hbuf[i, pl.ds(F - 16, 16)] + adbuf[i, :]
                e = jnp.where(e > 0.0, e, 0.2 * e)
                ee = jnp.exp(e)
                hbuf[i, pl.ds(F - 16, 16)] = ee
                dnums = lax.GatherDimensionNumbers(
                    offset_dims=(), collapsed_slice_dims=(0,),
                    start_index_map=(0,))
                for g in range(G - 1):
                    bidx = (lane + g * 16) >> shift
                    m = lax.gather(ee, bidx[:, None], dnums, (1,),
                                   mode=lax.GatherScatterMode.PROMISE_IN_BOUNDS)
                    hbuf[i, pl.ds(g * 16, 16)] = hbuf[i, pl.ds(g * 16, 16)] * m
                return carry2

            lax.fori_loop(0, _CH, edge_body, 0)
            pltpu.sync_copy(hbuf, acc.at[didx], add=True)
            return carry

        lax.fori_loop(0, nk, chunk_body, 0)
        plsc.subcore_barrier()
        pltpu.sync_copy(acc.at[pl.ds(s * _RPT, _RPT)],
                        out.at[c, pl.ds(s * _RPT, _RPT)])

    return edge_kernel


_edge_kernel_l1 = _make_edge_kernel(_F1, _HID)
_edge_kernel_l2 = _make_edge_kernel(_F2, _OUT)


def _tc1_body(x_ref, wext_ref, wd_ref, hext_ref, adt_ref):
    x = x_ref[...]
    hext_ref[...] = jnp.dot(x, wext_ref[...], preferred_element_type=jnp.float32)
    adt_ref[...] = jnp.dot(x, wd_ref[...], preferred_element_type=jnp.float32)


def _tc2_body(acc_ref, p1_ref, b1_ref, wext_ref, wd_ref, hext_ref, adt_ref):
    a = acc_ref[0] + acc_ref[1]
    num = a[:, :_D_IN]
    den = a[:, _D_IN:_D_IN + _HEADS]
    denr = jnp.dot(den, p1_ref[...], preferred_element_type=jnp.float32)
    x2 = jnp.maximum(num / (denr + 1e-16) + b1_ref[...], 0.0)
    hext_ref[...] = jnp.dot(x2, wext_ref[...], preferred_element_type=jnp.float32)
    adt_ref[...] = jnp.dot(x2, wd_ref[...], preferred_element_type=jnp.float32)


def _tc3_body(acc_ref, p2_ref, b2_ref, h_ref, lsm_ref):
    a = acc_ref[0] + acc_ref[1]
    fo = _HEADS * _OUT
    num = a[:, :fo]
    den = a[:, fo:fo + _HEADS]
    denr = jnp.dot(den, p2_ref[...], preferred_element_type=jnp.float32)
    h = num / (denr + 1e-16) + b2_ref[...]
    m = jnp.max(h, axis=1, keepdims=True)
    lse = jnp.log(jnp.sum(jnp.exp(h - m), axis=1, keepdims=True)) + m
    h_ref[...] = h
    lsm_ref[...] = h - lse


_R = 400


def _tc1(point, wext, wd):
    return pl.pallas_call(
        _tc1_body,
        grid=(_N // _R,),
        in_specs=[pl.BlockSpec((_R, _D_IN), lambda i: (i, 0)),
                  pl.BlockSpec((_D_IN, _F1), lambda i: (0, 0)),
                  pl.BlockSpec((_D_IN, 16), lambda i: (0, 0))],
        out_specs=[pl.BlockSpec((_R, _F1), lambda i: (i, 0)),
                   pl.BlockSpec((_R, 16), lambda i: (i, 0))],
        out_shape=[jax.ShapeDtypeStruct((_N, _F1), jnp.float32),
                   jax.ShapeDtypeStruct((_N, 16), jnp.float32)],
    )(point, wext, wd)


def _tc2(acc, p1, b1, wext2, wd2):
    return pl.pallas_call(
        _tc2_body,
        grid=(_N // _R,),
        in_specs=[pl.BlockSpec((_NC, _R, _F1), lambda i: (0, i, 0)),
                  pl.BlockSpec((_HEADS, _D_IN), lambda i: (0, 0)),
                  pl.BlockSpec((1, _D_IN), lambda i: (0, 0)),
                  pl.BlockSpec((_D_IN, _F2), lambda i: (0, 0)),
                  pl.BlockSpec((_D_IN, 16), lambda i: (0, 0))],
        out_specs=[pl.BlockSpec((_R, _F2), lambda i: (i, 0)),
                   pl.BlockSpec((_R, 16), lambda i: (i, 0))],
        out_shape=[jax.ShapeDtypeStruct((_N, _F2), jnp.float32),
                   jax.ShapeDtypeStruct((_N, 16), jnp.float32)],
    )(acc, p1, b1, wext2, wd2)


def _tc3(acc, p2, b2):
    fo = _HEADS * _OUT
    return pl.pallas_call(
        _tc3_body,
        grid=(_N // _R,),
        in_specs=[pl.BlockSpec((_NC, _R, _F2), lambda i: (0, i, 0)),
                  pl.BlockSpec((_HEADS, fo), lambda i: (0, 0)),
                  pl.BlockSpec((1, fo), lambda i: (0, 0))],
        out_specs=[pl.BlockSpec((_R, fo), lambda i: (i, 0)),
                   pl.BlockSpec((_R, fo), lambda i: (i, 0))],
        out_shape=[jax.ShapeDtypeStruct((_N, fo), jnp.float32),
                   jax.ShapeDtypeStruct((_N, fo), jnp.float32)],
    )(acc, p2, b2)


_OH1 = np.repeat(np.eye(_HEADS, dtype=np.float32), _HID, axis=0)
_OH2 = np.repeat(np.eye(_HEADS, dtype=np.float32), _OUT, axis=0)
_P1 = np.repeat(np.eye(_HEADS, dtype=np.float32), _HID, axis=1)
_P2 = np.repeat(np.eye(_HEADS, dtype=np.float32), _OUT, axis=1)


def kernel(point, edge_info, W1, a_src1, a_dst1, b1, W2, a_src2, a_dst2, b2):
    z8_128 = jnp.zeros((_D_IN, 8), jnp.float32)
    wext1 = jnp.concatenate(
        [W1, W1 @ (_OH1 * a_src1.reshape(-1, 1)), z8_128], axis=1)
    wd1 = jnp.concatenate(
        [W1 @ (_OH1 * a_dst1.reshape(-1, 1)), z8_128], axis=1)
    wext2 = jnp.concatenate(
        [W2, W2 @ (_OH2 * a_src2.reshape(-1, 1)), z8_128], axis=1)
    wd2 = jnp.concatenate(
        [W2 @ (_OH2 * a_dst2.reshape(-1, 1)), z8_128], axis=1)

    z1 = jnp.zeros((_RPT, _F1), jnp.float32)
    z2 = jnp.zeros((_RPT, _F2), jnp.float32)

    hext1, adt1 = _tc1(point, wext1, wd1)
    acc1 = _edge_kernel_l1(hext1, adt1, edge_info, z1)
    hext2, adt2 = _tc2(acc1, _P1, b1.reshape(1, -1), wext2, wd2)
    acc2 = _edge_kernel_l2(hext2, adt2, edge_info, z2)
    h, lsm = _tc3(acc2, _P2, b2.reshape(1, -1))
    return (h, lsm)


# speedup vs baseline: 83.8181x; 1.0196x over previous
import functools

import numpy as np
import jax
import jax.numpy as jnp
from jax import lax
from jax.experimental import pallas as pl
from jax.experimental.pallas import tpu as pltpu
from jax.experimental.pallas import tpu_sc as plsc

_N = 10000
_E = 320000
_D_IN = 128
_HID = 16
_OUT = 8
_HEADS = 8

_NC = 2
_NS = 16
_NW = _NC * _NS
_CH = 128
_NCHUNKS = _E // _CH
_RPT = 632
_NP = _NS * _RPT

_F1 = _D_IN + 16
_F2 = _HEADS * _OUT + 16


def _make_edge_kernel(F, c_per_head):
    G = F // 16
    mesh = plsc.VectorSubcoreMesh(core_axis_name="c", subcore_axis_name="s")

    @functools.partial(
        pl.kernel,
        mesh=mesh,
        compiler_params=pltpu.CompilerParams(use_tc_tiling_on_sc=False),
        out_type=jax.ShapeDtypeStruct((_NC, _NP, F), jnp.float32),
        scratch_types=[
            pltpu.VMEM((_CH,), jnp.int32),
            pltpu.VMEM((_CH,), jnp.int32),
            pltpu.VMEM((_CH, F), jnp.float32),
            pltpu.VMEM((_CH, 16), jnp.float32),
            pltpu.VMEM_SHARED((_NP, F), jnp.float32),
            pltpu.SemaphoreType.DMA,
            pltpu.SemaphoreType.DMA,
        ],
    )
    def edge_kernel(hext, adt, edges, zrows, out,
                    sidx, didx, hbuf, adbuf, acc, sem1, sem2):
        c = lax.axis_index("c")
        s = lax.axis_index("s")
        wid = c * _NS + s
        pltpu.sync_copy(zrows, acc.at[pl.ds(s * _RPT, _RPT)])
        plsc.subcore_barrier()

        nk = (_NCHUNKS + _NW - 1 - wid) // _NW

        def chunk_body(k, carry):
            base = (k * _NW + wid) * _CH
            pltpu.sync_copy(edges.at[0, pl.ds(base, _CH)], sidx)
            pltpu.sync_copy(edges.at[1, pl.ds(base, _CH)], didx)
            cp1 = pltpu.async_copy(hext.at[sidx], hbuf, sem1)
            cp2 = pltpu.async_copy(adt.at[didx], adbuf, sem2)
            cp1.wait()
            cp2.wait()

            def edge_body(i, carry2):
                e = hbuf[i, pl.ds(F - 16, 16)] + adbuf[i, :]
                e = jnp.where(e > 0.0, e, 0.2 * e)
                ee = jnp.exp(e)
                hbuf[i, pl.ds(F - 16, 16)] = ee
                for g in range(G - 1):
                    hbuf[i, pl.ds(g * 16, 16)] = hbuf[i, pl.ds(g * 16, 16)] * ee
                return carry2

            lax.fori_loop(0, _CH, edge_body, 0, unroll=4)
            pltpu.sync_copy(hbuf, acc.at[didx], add=True)
            return carry

        lax.fori_loop(0, nk, chunk_body, 0)
        plsc.subcore_barrier()
        pltpu.sync_copy(acc.at[pl.ds(s * _RPT, _RPT)],
                        out.at[c, pl.ds(s * _RPT, _RPT)])

    return edge_kernel


_edge_kernel_l1 = _make_edge_kernel(_F1, _HID)
_edge_kernel_l2 = _make_edge_kernel(_F2, _OUT)


def _tc1_body(x_ref, wext_ref, wd_ref, hext_ref, adt_ref):
    x = x_ref[...]
    hext_ref[...] = jnp.dot(x, wext_ref[...], preferred_element_type=jnp.float32)
    adt_ref[...] = jnp.dot(x, wd_ref[...], preferred_element_type=jnp.float32)


def _tc2_body(acc_ref, p1_ref, b1_ref, wext_ref, wd_ref, hext_ref, adt_ref):
    a = acc_ref[0] + acc_ref[1]
    num = a[:, :_D_IN]
    den = a[:, _D_IN:_D_IN + _HEADS]
    denr = jnp.dot(den, p1_ref[...], preferred_element_type=jnp.float32)
    x2 = jnp.maximum(num / (denr + 1e-16) + b1_ref[...], 0.0)
    hext_ref[...] = jnp.dot(x2, wext_ref[...], preferred_element_type=jnp.float32)
    adt_ref[...] = jnp.dot(x2, wd_ref[...], preferred_element_type=jnp.float32)


def _tc3_body(acc_ref, p2_ref, pm_ref, b2_ref, h_ref, lsm_ref):
    a = acc_ref[0] + acc_ref[1]
    fo = _HEADS * _OUT
    num = a[:, :fo]
    den = a[:, fo:fo + _HEADS]
    denr = jnp.dot(den, p2_ref[...], preferred_element_type=jnp.float32)
    h = jnp.dot(num / (denr + 1e-16), pm_ref[...],
                preferred_element_type=jnp.float32) + b2_ref[...]
    m = jnp.max(h, axis=1, keepdims=True)
    lse = jnp.log(jnp.sum(jnp.exp(h - m), axis=1, keepdims=True)) + m
    h_ref[...] = h
    lsm_ref[...] = h - lse


_R = 400


def _tc1(point, wext, wd):
    return pl.pallas_call(
        _tc1_body,
        grid=(_N // _R,),
        in_specs=[pl.BlockSpec((_R, _D_IN), lambda i: (i, 0)),
                  pl.BlockSpec((_D_IN, _F1), lambda i: (0, 0)),
                  pl.BlockSpec((_D_IN, 16), lambda i: (0, 0))],
        out_specs=[pl.BlockSpec((_R, _F1), lambda i: (i, 0)),
                   pl.BlockSpec((_R, 16), lambda i: (i, 0))],
        out_shape=[jax.ShapeDtypeStruct((_N, _F1), jnp.float32),
                   jax.ShapeDtypeStruct((_N, 16), jnp.float32)],
    )(point, wext, wd)


def _tc2(acc, p1, b1, wext2, wd2):
    return pl.pallas_call(
        _tc2_body,
        grid=(_N // _R,),
        in_specs=[pl.BlockSpec((_NC, _R, _F1), lambda i: (0, i, 0)),
                  pl.BlockSpec((_HEADS, _D_IN), lambda i: (0, 0)),
                  pl.BlockSpec((1, _D_IN), lambda i: (0, 0)),
                  pl.BlockSpec((_D_IN, _F2), lambda i: (0, 0)),
                  pl.BlockSpec((_D_IN, 16), lambda i: (0, 0))],
        out_specs=[pl.BlockSpec((_R, _F2), lambda i: (i, 0)),
                   pl.BlockSpec((_R, 16), lambda i: (i, 0))],
        out_shape=[jax.ShapeDtypeStruct((_N, _F2), jnp.float32),
                   jax.ShapeDtypeStruct((_N, 16), jnp.float32)],
    )(acc, p1, b1, wext2, wd2)


def _tc3(acc, p2, pm, b2):
    fo = _HEADS * _OUT
    return pl.pallas_call(
        _tc3_body,
        grid=(_N // _R,),
        in_specs=[pl.BlockSpec((_NC, _R, _F2), lambda i: (0, i, 0)),
                  pl.BlockSpec((_HEADS, fo), lambda i: (0, 0)),
                  pl.BlockSpec((fo, fo), lambda i: (0, 0)),
                  pl.BlockSpec((1, fo), lambda i: (0, 0))],
        out_specs=[pl.BlockSpec((_R, fo), lambda i: (i, 0)),
                   pl.BlockSpec((_R, fo), lambda i: (i, 0))],
        out_shape=[jax.ShapeDtypeStruct((_N, fo), jnp.float32),
                   jax.ShapeDtypeStruct((_N, fo), jnp.float32)],
    )(acc, p2, pm, b2)


_OH1 = np.repeat(np.eye(_HEADS, dtype=np.float32), _HID, axis=0)
_OH2 = np.repeat(np.eye(_HEADS, dtype=np.float32), _OUT, axis=0)
_IDX1 = np.array([(p % _HEADS) * _HID + p // _HEADS for p in range(_D_IN)])
_IDX2 = np.array([(p % _HEADS) * _OUT + p // _HEADS
                  for p in range(_HEADS * _OUT)])
_P1 = np.tile(np.eye(_HEADS, dtype=np.float32), (1, _HID))
_P2 = np.tile(np.eye(_HEADS, dtype=np.float32), (1, _OUT))
_PM = np.zeros((_HEADS * _OUT, _HEADS * _OUT), np.float32)
for _p in range(_HEADS * _OUT):
    _PM[_p, (_p % _HEADS) * _OUT + _p // _HEADS] = 1.0


def kernel(point, edge_info, W1, a_src1, a_dst1, b1, W2, a_src2, a_dst2, b2):
    as1 = W1 @ (_OH1 * a_src1.reshape(-1, 1))
    ad1 = W1 @ (_OH1 * a_dst1.reshape(-1, 1))
    wext1 = jnp.concatenate([W1[:, _IDX1], as1, as1], axis=1)
    wd1 = jnp.concatenate([ad1, ad1], axis=1)
    W2p = W2[_IDX1, :]
    as2 = W2p @ (_OH2 * a_src2.reshape(-1, 1))
    ad2 = W2p @ (_OH2 * a_dst2.reshape(-1, 1))
    wext2 = jnp.concatenate([W2p[:, _IDX2], as2, as2], axis=1)
    wd2 = jnp.concatenate([ad2, ad2], axis=1)

    z1 = jnp.zeros((_RPT, _F1), jnp.float32)
    z2 = jnp.zeros((_RPT, _F2), jnp.float32)

    hext1, adt1 = _tc1(point, wext1, wd1)
    acc1 = _edge_kernel_l1(hext1, adt1, edge_info, z1)
    hext2, adt2 = _tc2(acc1, _P1, b1[_IDX1].reshape(1, -1), wext2, wd2)
    acc2 = _edge_kernel_l2(hext2, adt2, edge_info, z2)
    h, lsm = _tc3(acc2, _P2, _PM, b2.reshape(1, -1))
    return (h, lsm)
